# baseline (speedup 1.0000x reference)
import functools

import jax
import jax.numpy as jnp
from jax import lax
from jax.experimental import pallas as pl
from jax.experimental.pallas import tpu as pltpu
from jax.experimental.pallas import tpu_sc as plsc

N_PRE = 5
N_CLS = 4
N_SUF = 68
N_TOK = 77
D = 512

PRE_W = N_PRE * D
CLS_W = N_CLS * D
SUF_W = N_SUF * D
TMPL_W = PRE_W + SUF_W
ROW_W = N_TOK * D

NC = 2
NS = 16
NW = NC * NS
B = 1024
RPW = B // NW
CHUNK = 8


def _sc_body(tmpl_hbm, cls_hbm, lab_hbm, sel_hbm, out_hbm,
             tmpl_v, lab_v, sel_v, cls_v, sem):
    wid = lax.axis_index("s") * NC + lax.axis_index("c")
    base = wid * RPW
    pltpu.sync_copy(tmpl_hbm, tmpl_v)
    pltpu.sync_copy(lab_hbm.at[pl.ds(base, RPW)], lab_v)
    pltpu.sync_copy(sel_hbm.at[pl.ds(base, RPW)], sel_v)
    for c in range(RPW // CHUNK):
        pltpu.async_copy(
            cls_hbm.at[lab_v.at[pl.ds(c * CHUNK, CHUNK)]], cls_v, sem
        ).wait()
        for j in range(CHUNK):
            r = c * CHUNK + j
            row = base + r
            k, lane = divmod(r, 16)
            s = sel_v[pl.ds(k * 16, 16)][lane]
            pltpu.sync_copy(tmpl_v.at[s, pl.ds(0, PRE_W)],
                            out_hbm.at[row, pl.ds(0, PRE_W)])
            pltpu.sync_copy(cls_v.at[j],
                            out_hbm.at[row, pl.ds(PRE_W, CLS_W)])
            pltpu.sync_copy(tmpl_v.at[s, pl.ds(PRE_W, SUF_W)],
                            out_hbm.at[row, pl.ds(PRE_W + CLS_W, SUF_W)])


@functools.partial(jax.jit, static_argnums=())
def _sc_call(tmpl, cls_flat, label, sel):
    mesh = plsc.VectorSubcoreMesh(
        core_axis_name="c", subcore_axis_name="s",
        num_cores=NC, num_subcores=NS)
    return pl.kernel(
        _sc_body,
        out_type=jax.ShapeDtypeStruct((B, ROW_W), jnp.float32),
        mesh=mesh,
        scratch_types=[
            pltpu.VMEM((2, TMPL_W), jnp.float32),
            pltpu.VMEM((RPW,), jnp.int32),
            pltpu.VMEM((RPW,), jnp.int32),
            pltpu.VMEM((CHUNK, CLS_W), jnp.float32),
            pltpu.SemaphoreType.DMA,
        ],
    )(tmpl, cls_flat, label, sel)


def kernel(label, cluster, cls_ctx, token_prefix, token_suffix,
           token_prefix_cluster, token_suffix_cluster):
    sel = cluster.astype(jnp.int32)
    tmpl = jnp.stack([
        jnp.concatenate([token_prefix[0].reshape(-1),
                         token_suffix[0].reshape(-1)]),
        jnp.concatenate([token_prefix_cluster[0].reshape(-1),
                         token_suffix_cluster[0].reshape(-1)]),
    ])
    cls_flat = cls_ctx.reshape(cls_ctx.shape[0], CLS_W)
    out = _sc_call(tmpl, cls_flat, label, sel)
    return out.reshape(B, N_TOK, D)


# speedup vs baseline: 1.0051x; 1.0051x over previous
import functools

import jax
import jax.numpy as jnp
from jax import lax
from jax.experimental import pallas as pl
from jax.experimental.pallas import tpu as pltpu
from jax.experimental.pallas import tpu_sc as plsc

N_PRE = 5
N_CLS = 4
N_SUF = 68
N_TOK = 77
D = 512

PRE_W = N_PRE * D
CLS_W = N_CLS * D
SUF_W = N_SUF * D
TMPL_W = PRE_W + SUF_W
ROW_W = N_TOK * D

NC = 2
NS = 16
NW = NC * NS
B = 1024
RPW = B // NW
CHUNK = 8


def _sc_body(tmpl_hbm, cls_hbm, lab_hbm, sel_hbm, out_hbm,
             tmpl_v, lab_v, sel_v, cls_v, gsem, ssem):
    wid = lax.axis_index("s") * NC + lax.axis_index("c")
    base = wid * RPW
    pltpu.sync_copy(tmpl_hbm, tmpl_v)
    pltpu.sync_copy(lab_hbm.at[pl.ds(base, RPW)], lab_v)
    pltpu.sync_copy(sel_hbm.at[pl.ds(base, RPW)], sel_v)
    nch = RPW // CHUNK
    gathers = [None] * nch
    cls_cps = [[] for _ in range(nch)]
    tmpl_cps = []
    gathers[0] = pltpu.async_copy(
        cls_hbm.at[lab_v.at[pl.ds(0, CHUNK)]], cls_v.at[0], gsem)
    for c in range(nch):
        if c + 1 < nch:
            for cp in (cls_cps[c - 1] if c >= 1 else ()):
                cp.wait()
            gathers[c + 1] = pltpu.async_copy(
                cls_hbm.at[lab_v.at[pl.ds((c + 1) * CHUNK, CHUNK)]],
                cls_v.at[(c + 1) % 2], gsem)
        gathers[c].wait()
        for j in range(CHUNK):
            r = c * CHUNK + j
            row = base + r
            k, lane = divmod(r, 16)
            s = sel_v[pl.ds(k * 16, 16)][lane]
            tmpl_cps.append(pltpu.async_copy(
                tmpl_v.at[s, pl.ds(0, PRE_W)],
                out_hbm.at[row, pl.ds(0, PRE_W)], ssem))
            cls_cps[c].append(pltpu.async_copy(
                cls_v.at[c % 2, j],
                out_hbm.at[row, pl.ds(PRE_W, CLS_W)], ssem))
            tmpl_cps.append(pltpu.async_copy(
                tmpl_v.at[s, pl.ds(PRE_W, SUF_W)],
                out_hbm.at[row, pl.ds(PRE_W + CLS_W, SUF_W)], ssem))
    for cp in cls_cps[nch - 2] + cls_cps[nch - 1] + tmpl_cps:
        cp.wait()


@functools.partial(jax.jit, static_argnums=())
def _sc_call(tmpl, cls_flat, label, sel):
    mesh = plsc.VectorSubcoreMesh(
        core_axis_name="c", subcore_axis_name="s",
        num_cores=NC, num_subcores=NS)
    return pl.kernel(
        _sc_body,
        out_type=jax.ShapeDtypeStruct((B, ROW_W), jnp.float32),
        mesh=mesh,
        scratch_types=[
            pltpu.VMEM((2, TMPL_W), jnp.float32),
            pltpu.VMEM((RPW,), jnp.int32),
            pltpu.VMEM((RPW,), jnp.int32),
            pltpu.VMEM((2, CHUNK, CLS_W), jnp.float32),
            pltpu.SemaphoreType.DMA,
            pltpu.SemaphoreType.DMA,
        ],
    )(tmpl, cls_flat, label, sel)


def kernel(label, cluster, cls_ctx, token_prefix, token_suffix,
           token_prefix_cluster, token_suffix_cluster):
    sel = cluster.astype(jnp.int32)
    tmpl = jnp.stack([
        jnp.concatenate([token_prefix[0].reshape(-1),
                         token_suffix[0].reshape(-1)]),
        jnp.concatenate([token_prefix_cluster[0].reshape(-1),
                         token_suffix_cluster[0].reshape(-1)]),
    ])
    cls_flat = cls_ctx.reshape(cls_ctx.shape[0], CLS_W)
    out = _sc_call(tmpl, cls_flat, label, sel)
    return out.reshape(B, N_TOK, D)


# speedup vs baseline: 1.0365x; 1.0312x over previous
import jax
import jax.numpy as jnp
from jax import lax
from jax.experimental import pallas as pl
from jax.experimental.pallas import tpu as pltpu
from jax.experimental.pallas import tpu_sc as plsc

N_PRE = 5
N_CLS = 4
N_SUF = 68
N_TOK = 77
D = 512

NC = 2
NS = 16
NW = NC * NS
B = 1024
RPW = B // NW
CHUNK = 8


def _sc_body(tmpl_hbm, cls_hbm, lab_hbm, sel_hbm, out_hbm,
             tmpl_v, lab_v, sel_v, cls_v, gsem, ssem):
    wid = lax.axis_index("s") * NC + lax.axis_index("c")
    base = wid * RPW
    pltpu.sync_copy(tmpl_hbm, tmpl_v)
    pltpu.sync_copy(lab_hbm.at[pl.ds(base, RPW)], lab_v)
    pltpu.sync_copy(sel_hbm.at[pl.ds(base, RPW)], sel_v)
    nch = RPW // CHUNK
    gathers = [None] * nch
    cls_cps = [[] for _ in range(nch)]
    tmpl_cps = []
    gathers[0] = pltpu.async_copy(
        cls_hbm.at[lab_v.at[pl.ds(0, CHUNK)]], cls_v.at[0], gsem)
    for c in range(nch):
        if c + 1 < nch:
            for cp in (cls_cps[c - 1] if c >= 1 else ()):
                cp.wait()
            gathers[c + 1] = pltpu.async_copy(
                cls_hbm.at[lab_v.at[pl.ds((c + 1) * CHUNK, CHUNK)]],
                cls_v.at[(c + 1) % 2], gsem)
        gathers[c].wait()
        for j in range(CHUNK):
            r = c * CHUNK + j
            row = base + r
            k, lane = divmod(r, 16)
            s = sel_v[pl.ds(k * 16, 16)][lane]
            tmpl_cps.append(pltpu.async_copy(
                tmpl_v.at[s, pl.ds(0, N_PRE)],
                out_hbm.at[row, pl.ds(0, N_PRE)], ssem))
            cls_cps[c].append(pltpu.async_copy(
                cls_v.at[c % 2, j],
                out_hbm.at[row, pl.ds(N_PRE, N_CLS)], ssem))
            tmpl_cps.append(pltpu.async_copy(
                tmpl_v.at[s, pl.ds(N_PRE, N_SUF)],
                out_hbm.at[row, pl.ds(N_PRE + N_CLS, N_SUF)], ssem))
    for cp in cls_cps[nch - 2] + cls_cps[nch - 1] + tmpl_cps:
        cp.wait()


@jax.jit
def _sc_call(tmpl, cls_ctx, label, sel):
    mesh = plsc.VectorSubcoreMesh(
        core_axis_name="c", subcore_axis_name="s",
        num_cores=NC, num_subcores=NS)
    return pl.kernel(
        _sc_body,
        out_type=jax.ShapeDtypeStruct((B, N_TOK, D), jnp.float32),
        mesh=mesh,
        compiler_params=pltpu.CompilerParams(use_tc_tiling_on_sc=False),
        scratch_types=[
            pltpu.VMEM((2, N_PRE + N_SUF, D), jnp.float32),
            pltpu.VMEM((RPW,), jnp.int32),
            pltpu.VMEM((RPW,), jnp.int32),
            pltpu.VMEM((2, CHUNK, N_CLS, D), jnp.float32),
            pltpu.SemaphoreType.DMA,
            pltpu.SemaphoreType.DMA,
        ],
    )(tmpl, cls_ctx, label, sel)


def kernel(label, cluster, cls_ctx, token_prefix, token_suffix,
           token_prefix_cluster, token_suffix_cluster):
    sel = cluster.astype(jnp.int32)
    tmpl = jnp.stack([
        jnp.concatenate([token_prefix[0], token_suffix[0]], axis=0),
        jnp.concatenate([token_prefix_cluster[0], token_suffix_cluster[0]],
                        axis=0),
    ])
    return _sc_call(tmpl, cls_ctx, label, sel)


# speedup vs baseline: 4.0718x; 3.9284x over previous
import jax
import jax.numpy as jnp
from jax import lax
from jax.experimental import pallas as pl
from jax.experimental.pallas import tpu as pltpu
from jax.experimental.pallas import tpu_sc as plsc

N_PRE = 5
N_CLS = 4
N_SUF = 68
N_TOK = 77
D = 512

NC = 2
NS = 16
NW = NC * NS
B = 1024
RPW = B // NW
CH = 8
TAIL = 5
PATCH_RPS = 32


def _insert_cls(cls_v, j, b0r_v, b1r_v, s, p):
    def body0(i, _):
        rr = i // 4
        wb = (i % 4) * 128
        for k in range(8):
            w = pl.ds(wb + k * 16, 16)
            b0r_v[s, p, 5 + rr, w] = cls_v[j, rr, w]
        return 0

    def body1(i, _):
        wb = i * 128
        for k in range(8):
            w = pl.ds(wb + k * 16, 16)
            b1r_v[s, 0, w] = cls_v[j, 3, w]
        return 0

    lax.fori_loop(0, 12, body0, 0, unroll=False)
    lax.fori_loop(0, 4, body1, 0, unroll=False)


def _sc_body(tmpl_hbm, cls_hbm, lab_hbm, sel_hbm, out_hbm,
             suf_v, b0r_v, b1r_v, cls_v, lab_v, sel_v, gsem, ssem):
    wid = lax.axis_index("s") * NC + lax.axis_index("c")
    base = wid * RPW
    for v in (0, 1):
        pltpu.sync_copy(tmpl_hbm.at[v, pl.ds(8, 64)], suf_v.at[v])
        pltpu.sync_copy(tmpl_hbm.at[v, pl.ds(8, 8)], b1r_v.at[v])
        for p in (0, 1):
            pltpu.sync_copy(tmpl_hbm.at[v, pl.ds(0, 8)], b0r_v.at[v, p])
    pltpu.sync_copy(lab_hbm.at[pl.ds(base, RPW)], lab_v)
    pltpu.sync_copy(sel_hbm.at[pl.ds(base, RPW)], sel_v)

    nch = RPW // CH
    gathers = [None] * nch
    ring_cps = [None] * RPW
    gathers[0] = pltpu.async_copy(
        cls_hbm.at[lab_v.at[pl.ds(0, CH)]], cls_v, gsem)
    for c in range(nch):
        gathers[c].wait()
        for j in range(CH):
            r = c * CH + j
            row = base + r
            p = r % 2
            k, lane = divmod(r, 16)
            s = sel_v[pl.ds(k * 16, 16)][lane]
            if r >= 2:
                ring_cps[r - 2][0].wait()
                ring_cps[r - 2][2].wait()
            if r >= 1:
                ring_cps[r - 1][1].wait()
            _insert_cls(cls_v, j, b0r_v, b1r_v, s, p)
            ring_cps[r] = [
                pltpu.async_copy(b0r_v.at[s, p],
                                 out_hbm.at[row, pl.ds(0, 8)], ssem),
                pltpu.async_copy(b1r_v.at[s],
                                 out_hbm.at[row, pl.ds(8, 8)], ssem),
                pltpu.async_copy(suf_v.at[s, pl.ds(8, 56)],
                                 out_hbm.at[row, pl.ds(16, 56)], ssem),
            ]
        if c + 1 < nch:
            gathers[c + 1] = pltpu.async_copy(
                cls_hbm.at[lab_v.at[pl.ds((c + 1) * CH, CH)]], cls_v, gsem)
    for r in (RPW - 2, RPW - 1):
        ring_cps[r][0].wait()
        ring_cps[r][2].wait()
    ring_cps[RPW - 1][1].wait()


def _patch_body(sel_ref, tails_ref, in_ref, out_ref, *sems):
    del in_ref
    i = pl.program_id(0)
    cps = []
    for j in range(PATCH_RPS):
        row = i * PATCH_RPS + j
        s = sel_ref[row]
        cps.append(pltpu.make_async_copy(
            tails_ref.at[s],
            out_ref.at[row, pl.ds(N_TOK - TAIL, TAIL)],
            sems[j % 4]))
    for cp in cps:
        cp.start()
    for cp in cps:
        cp.wait()


@jax.jit
def _call(tmpl, tails, cls_ctx, label, sel):
    mesh = plsc.VectorSubcoreMesh(
        core_axis_name="c", subcore_axis_name="s",
        num_cores=NC, num_subcores=NS)
    out = pl.kernel(
        _sc_body,
        out_type=jax.ShapeDtypeStruct((B, N_TOK, D), jnp.float32),
        mesh=mesh,
        scratch_types=[
            pltpu.VMEM((2, 64, D), jnp.float32),
            pltpu.VMEM((2, 2, 8, D), jnp.float32),
            pltpu.VMEM((2, 8, D), jnp.float32),
            pltpu.VMEM((CH, N_CLS, D), jnp.float32),
            pltpu.VMEM((RPW,), jnp.int32),
            pltpu.VMEM((RPW,), jnp.int32),
            pltpu.SemaphoreType.DMA,
            pltpu.SemaphoreType.DMA,
        ],
    )(tmpl, cls_ctx, label, sel)
    out = pl.pallas_call(
        _patch_body,
        grid=(B // PATCH_RPS,),
        in_specs=[
            pl.BlockSpec(memory_space=pltpu.SMEM),
            pl.BlockSpec(memory_space=pltpu.VMEM),
            pl.BlockSpec(memory_space=pl.ANY),
        ],
        out_specs=pl.BlockSpec(memory_space=pl.ANY),
        out_shape=jax.ShapeDtypeStruct((B, N_TOK, D), jnp.float32),
        input_output_aliases={2: 0},
        scratch_shapes=[pltpu.SemaphoreType.DMA] * 4,
    )(sel, tails, out)
    return out


def kernel(label, cluster, cls_ctx, token_prefix, token_suffix,
           token_prefix_cluster, token_suffix_cluster):
    sel = cluster.astype(jnp.int32)
    zmid = jnp.zeros((N_CLS, D), dtype=token_prefix.dtype)
    tmpl = jnp.stack([
        jnp.concatenate([token_prefix[0], zmid, token_suffix[0]], axis=0),
        jnp.concatenate([token_prefix_cluster[0], zmid,
                         token_suffix_cluster[0]], axis=0),
    ])
    tails = jnp.stack([token_suffix[0, N_SUF - TAIL:],
                       token_suffix_cluster[0, N_SUF - TAIL:]])
    return _call(tmpl, tails, cls_ctx, label, sel)


# speedup vs baseline: 4.0844x; 1.0031x over previous
import jax
import jax.numpy as jnp
from jax import lax
from jax.experimental import pallas as pl
from jax.experimental.pallas import tpu as pltpu
from jax.experimental.pallas import tpu_sc as plsc

N_PRE = 5
N_CLS = 4
N_SUF = 68
N_TOK = 77
D = 512

NC = 2
NS = 16
NW = NC * NS
B = 1024
RPW = B // NW
CH = 8
TAIL = 5
PATCH_RPS = 32


def _insert_cls(cls_v, j, b0r_v, b1r_v, s, p):
    def body0(i, _):
        rr = i // 4
        wb = (i % 4) * 128
        for k in range(8):
            w = pl.ds(wb + k * 16, 16)
            b0r_v[s, p, 5 + rr, w] = cls_v[j, rr, w]
        return 0

    def body1(i, _):
        wb = i * 128
        for k in range(8):
            w = pl.ds(wb + k * 16, 16)
            b1r_v[s, 0, w] = cls_v[j, 3, w]
        return 0

    lax.fori_loop(0, 12, body0, 0, unroll=False)
    lax.fori_loop(0, 4, body1, 0, unroll=False)


def _sc_body(tmpl_hbm, cls_hbm, lab_hbm, sel_hbm, out_hbm,
             b0r_v, b1r_v, cls_v, lab_v, sel_v, gsem, ssem):
    wid = lax.axis_index("s") * NC + lax.axis_index("c")
    base = wid * RPW
    for v in (0, 1):
        pltpu.sync_copy(tmpl_hbm.at[v, pl.ds(8, 8)], b1r_v.at[v])
        for p in (0, 1):
            pltpu.sync_copy(tmpl_hbm.at[v, pl.ds(0, 8)], b0r_v.at[v, p])
    pltpu.sync_copy(lab_hbm.at[pl.ds(base, RPW)], lab_v)
    pltpu.sync_copy(sel_hbm.at[pl.ds(base, RPW)], sel_v)

    nch = RPW // CH
    gathers = [None] * nch
    ring_cps = [None] * RPW
    gathers[0] = pltpu.async_copy(
        cls_hbm.at[lab_v.at[pl.ds(0, CH)]], cls_v, gsem)
    for c in range(nch):
        gathers[c].wait()
        for j in range(CH):
            r = c * CH + j
            row = base + r
            p = r % 2
            k, lane = divmod(r, 16)
            s = sel_v[pl.ds(k * 16, 16)][lane]
            if r >= 2:
                ring_cps[r - 2][0].wait()
            if r >= 1:
                ring_cps[r - 1][1].wait()
            _insert_cls(cls_v, j, b0r_v, b1r_v, s, p)
            ring_cps[r] = [
                pltpu.async_copy(b0r_v.at[s, p],
                                 out_hbm.at[row, pl.ds(0, 8)], ssem),
                pltpu.async_copy(b1r_v.at[s],
                                 out_hbm.at[row, pl.ds(8, 8)], ssem),
            ]
        if c + 1 < nch:
            gathers[c + 1] = pltpu.async_copy(
                cls_hbm.at[lab_v.at[pl.ds((c + 1) * CH, CH)]], cls_v, gsem)
    for r in (RPW - 2, RPW - 1):
        ring_cps[r][0].wait()
    ring_cps[RPW - 1][1].wait()


def _patch_body(sel_ref, tmpl_ref, in_ref, out_ref, *sems):
    del in_ref
    i = pl.program_id(0)
    cps = []
    for j in range(PATCH_RPS):
        row = i * PATCH_RPS + j
        s = sel_ref[row]
        cps.append(pltpu.make_async_copy(
            tmpl_ref.at[s, pl.ds(16, N_TOK - 16)],
            out_ref.at[row, pl.ds(16, N_TOK - 16)],
            sems[j % 4]))
    for cp in cps:
        cp.start()
    for cp in cps:
        cp.wait()


@jax.jit
def _call(tmpl, cls_ctx, label, sel):
    mesh = plsc.VectorSubcoreMesh(
        core_axis_name="c", subcore_axis_name="s",
        num_cores=NC, num_subcores=NS)
    out = pl.kernel(
        _sc_body,
        out_type=jax.ShapeDtypeStruct((B, N_TOK, D), jnp.float32),
        mesh=mesh,
        scratch_types=[
            pltpu.VMEM((2, 2, 8, D), jnp.float32),
            pltpu.VMEM((2, 8, D), jnp.float32),
            pltpu.VMEM((CH, N_CLS, D), jnp.float32),
            pltpu.VMEM((RPW,), jnp.int32),
            pltpu.VMEM((RPW,), jnp.int32),
            pltpu.SemaphoreType.DMA,
            pltpu.SemaphoreType.DMA,
        ],
    )(tmpl, cls_ctx, label, sel)
    out = pl.pallas_call(
        _patch_body,
        grid=(B // PATCH_RPS,),
        in_specs=[
            pl.BlockSpec(memory_space=pltpu.SMEM),
            pl.BlockSpec(memory_space=pltpu.VMEM),
            pl.BlockSpec(memory_space=pl.ANY),
        ],
        out_specs=pl.BlockSpec(memory_space=pl.ANY),
        out_shape=jax.ShapeDtypeStruct((B, N_TOK, D), jnp.float32),
        input_output_aliases={2: 0},
        scratch_shapes=[pltpu.SemaphoreType.DMA] * 4,
    )(sel, tmpl, out)
    return out


def kernel(label, cluster, cls_ctx, token_prefix, token_suffix,
           token_prefix_cluster, token_suffix_cluster):
    sel = cluster.astype(jnp.int32)
    zmid = jnp.zeros((N_CLS, D), dtype=token_prefix.dtype)
    tmpl = jnp.stack([
        jnp.concatenate([token_prefix[0], zmid, token_suffix[0]], axis=0),
        jnp.concatenate([token_prefix_cluster[0], zmid,
                         token_suffix_cluster[0]], axis=0),
    ])
    return _call(tmpl, cls_ctx, label, sel)


# speedup vs baseline: 4.1611x; 1.0188x over previous
import jax
import jax.numpy as jnp
from jax import lax
from jax.experimental import pallas as pl
from jax.experimental.pallas import tpu as pltpu
from jax.experimental.pallas import tpu_sc as plsc

N_PRE = 5
N_CLS = 4
N_SUF = 68
N_TOK = 77
D = 512

NC = 2
NS = 16
NW = NC * NS
B = 1024
RPW = B // NW
CH = 8
TAIL = 5
PATCH_RPS = 32


def _insert_cls(cls_v, j, hd_v, s, p):
    def body0(i, _):
        rr = i // 4
        wb = (i % 4) * 128
        for k in range(8):
            w = pl.ds(wb + k * 16, 16)
            hd_v[s, p, 5 + rr, w] = cls_v[j, rr, w]
        return 0

    lax.fori_loop(0, 16, body0, 0, unroll=False)


def _sc_body(tmpl_hbm, cls_hbm, lab_hbm, sel_hbm, out_hbm,
             suf_v, hd_v, cls_v, lab_v, sel_v, gsem, ssem):
    wid = lax.axis_index("s") * NC + lax.axis_index("c")
    base = wid * RPW
    for v in (0, 1):
        pltpu.sync_copy(tmpl_hbm.at[v, pl.ds(8, 64)], suf_v.at[v])
        for p in (0, 1):
            pltpu.sync_copy(tmpl_hbm.at[v, pl.ds(0, 16)], hd_v.at[v, p])
    pltpu.sync_copy(lab_hbm.at[pl.ds(base, RPW)], lab_v)
    pltpu.sync_copy(sel_hbm.at[pl.ds(base, RPW)], sel_v)

    nch = RPW // CH
    gathers = [None] * nch
    ring_cps = [None] * RPW
    gathers[0] = pltpu.async_copy(
        cls_hbm.at[lab_v.at[pl.ds(0, CH)]], cls_v, gsem)
    for c in range(nch):
        gathers[c].wait()
        for j in range(CH):
            r = c * CH + j
            row = base + r
            p = r % 2
            k, lane = divmod(r, 16)
            s = sel_v[pl.ds(k * 16, 16)][lane]
            if r >= 2:
                ring_cps[r - 2][0].wait()
                ring_cps[r - 2][1].wait()
            _insert_cls(cls_v, j, hd_v, s, p)
            ring_cps[r] = [
                pltpu.async_copy(hd_v.at[s, p],
                                 out_hbm.at[row, pl.ds(0, 16)], ssem),
                pltpu.async_copy(suf_v.at[s, pl.ds(8, 56)],
                                 out_hbm.at[row, pl.ds(16, 56)], ssem),
            ]
        if c + 1 < nch:
            gathers[c + 1] = pltpu.async_copy(
                cls_hbm.at[lab_v.at[pl.ds((c + 1) * CH, CH)]], cls_v, gsem)
    for r in (RPW - 2, RPW - 1):
        ring_cps[r][0].wait()
        ring_cps[r][1].wait()


def _patch_body(sel_ref, tails_ref, in_ref, out_ref, *sems):
    del in_ref
    i = pl.program_id(0)
    cps = []
    for j in range(PATCH_RPS):
        row = i * PATCH_RPS + j
        s = sel_ref[row]
        cps.append(pltpu.make_async_copy(
            tails_ref.at[s],
            out_ref.at[row, pl.ds(N_TOK - TAIL, TAIL)],
            sems[j % 4]))
    for cp in cps:
        cp.start()
    for cp in cps:
        cp.wait()


@jax.jit
def _call(tmpl, tails, cls_ctx, label, sel):
    mesh = plsc.VectorSubcoreMesh(
        core_axis_name="c", subcore_axis_name="s",
        num_cores=NC, num_subcores=NS)
    out = pl.kernel(
        _sc_body,
        out_type=jax.ShapeDtypeStruct((B, N_TOK, D), jnp.float32),
        mesh=mesh,
        scratch_types=[
            pltpu.VMEM((2, 64, D), jnp.float32),
            pltpu.VMEM((2, 2, 16, D), jnp.float32),
            pltpu.VMEM((CH, N_CLS, D), jnp.float32),
            pltpu.VMEM((RPW,), jnp.int32),
            pltpu.VMEM((RPW,), jnp.int32),
            pltpu.SemaphoreType.DMA,
            pltpu.SemaphoreType.DMA,
        ],
    )(tmpl, cls_ctx, label, sel)
    out = pl.pallas_call(
        _patch_body,
        grid=(B // PATCH_RPS,),
        in_specs=[
            pl.BlockSpec(memory_space=pltpu.SMEM),
            pl.BlockSpec(memory_space=pltpu.VMEM),
            pl.BlockSpec(memory_space=pl.ANY),
        ],
        out_specs=pl.BlockSpec(memory_space=pl.ANY),
        out_shape=jax.ShapeDtypeStruct((B, N_TOK, D), jnp.float32),
        input_output_aliases={2: 0},
        scratch_shapes=[pltpu.SemaphoreType.DMA] * 4,
    )(sel, tails, out)
    return out


def kernel(label, cluster, cls_ctx, token_prefix, token_suffix,
           token_prefix_cluster, token_suffix_cluster):
    sel = cluster.astype(jnp.int32)
    zmid = jnp.zeros((N_CLS, D), dtype=token_prefix.dtype)
    tmpl = jnp.stack([
        jnp.concatenate([token_prefix[0], zmid, token_suffix[0]], axis=0),
        jnp.concatenate([token_prefix_cluster[0], zmid,
                         token_suffix_cluster[0]], axis=0),
    ])
    tails = jnp.stack([token_suffix[0, N_SUF - TAIL:],
                       token_suffix_cluster[0, N_SUF - TAIL:]])
    return _call(tmpl, tails, cls_ctx, label, sel)


# speedup vs baseline: 4.1701x; 1.0022x over previous
import jax
import jax.numpy as jnp
from jax import lax
from jax.experimental import pallas as pl
from jax.experimental.pallas import tpu as pltpu
from jax.experimental.pallas import tpu_sc as plsc

N_PRE = 5
N_CLS = 4
N_SUF = 68
N_TOK = 77
D = 512

NC = 2
NS = 16
NW = NC * NS
B = 1024
RPW = B // NW
CH = 8
TAIL = 5
PATCH_RPS = 32


def _insert_cls(cls_v, j, hd_v, s, p):
    def body0(i, _):
        rr = i // 4
        wb = (i % 4) * 128
        for k in range(8):
            w = pl.ds(wb + k * 16, 16)
            hd_v[s, p, 5 + rr, w] = cls_v[j, rr, w]
        return 0

    lax.fori_loop(0, 16, body0, 0, unroll=False)


def _sc_body(tmpl_hbm, cls_hbm, lab_hbm, sel_hbm, out_hbm,
             suf_v, hd_v, cls_v, lab_v, sel_v, gsem, ssem):
    wid = lax.axis_index("s") * NC + lax.axis_index("c")
    base = wid * RPW
    for v in (0, 1):
        pltpu.sync_copy(tmpl_hbm.at[v, pl.ds(8, 64)], suf_v.at[v])
        for p in (0, 1):
            pltpu.sync_copy(tmpl_hbm.at[v, pl.ds(0, 16)], hd_v.at[v, p])
    pltpu.sync_copy(lab_hbm.at[pl.ds(base, RPW)], lab_v)
    pltpu.sync_copy(sel_hbm.at[pl.ds(base, RPW)], sel_v)

    nch = RPW // CH
    gathers = [None] * nch
    ring_cps = [None] * RPW
    gathers[0] = pltpu.async_copy(
        cls_hbm.at[lab_v.at[pl.ds(0, CH)]], cls_v, gsem)
    for c in range(nch):
        gathers[c].wait()
        for j in range(CH):
            r = c * CH + j
            row = base + r
            p = r % 2
            k, lane = divmod(r, 16)
            s = sel_v[pl.ds(k * 16, 16)][lane]
            if r >= 2:
                ring_cps[r - 2][0].wait()
                ring_cps[r - 2][1].wait()
            _insert_cls(cls_v, j, hd_v, s, p)
            ring_cps[r] = [
                pltpu.async_copy(hd_v.at[s, p],
                                 out_hbm.at[row, pl.ds(0, 16)], ssem),
                pltpu.async_copy(suf_v.at[s, pl.ds(8, 56)],
                                 out_hbm.at[row, pl.ds(16, 56)], ssem),
            ]
        if c + 1 < nch:
            gathers[c + 1] = pltpu.async_copy(
                cls_hbm.at[lab_v.at[pl.ds((c + 1) * CH, CH)]], cls_v, gsem)
    for r in (RPW - 2, RPW - 1):
        ring_cps[r][0].wait()
        ring_cps[r][1].wait()


def _patch_body(sel_ref, tails_ref, in_ref, out_ref, *sems):
    del in_ref
    i = pl.program_id(0)
    cps = []
    for j in range(PATCH_RPS):
        row = i * PATCH_RPS + j
        s = sel_ref[row]
        cps.append(pltpu.make_async_copy(
            tails_ref.at[s, pl.ds(N_TOK - TAIL, TAIL)],
            out_ref.at[row, pl.ds(N_TOK - TAIL, TAIL)],
            sems[j % 4]))
    for cp in cps:
        cp.start()
    for cp in cps:
        cp.wait()


@jax.jit
def _call(tmpl, cls_ctx, label, sel):
    mesh = plsc.VectorSubcoreMesh(
        core_axis_name="c", subcore_axis_name="s",
        num_cores=NC, num_subcores=NS)
    out = pl.kernel(
        _sc_body,
        out_type=jax.ShapeDtypeStruct((B, N_TOK, D), jnp.float32),
        mesh=mesh,
        scratch_types=[
            pltpu.VMEM((2, 64, D), jnp.float32),
            pltpu.VMEM((2, 2, 16, D), jnp.float32),
            pltpu.VMEM((CH, N_CLS, D), jnp.float32),
            pltpu.VMEM((RPW,), jnp.int32),
            pltpu.VMEM((RPW,), jnp.int32),
            pltpu.SemaphoreType.DMA,
            pltpu.SemaphoreType.DMA,
        ],
    )(tmpl, cls_ctx, label, sel)
    out = pl.pallas_call(
        _patch_body,
        grid=(B // PATCH_RPS,),
        in_specs=[
            pl.BlockSpec(memory_space=pltpu.SMEM),
            pl.BlockSpec(memory_space=pltpu.VMEM),
            pl.BlockSpec(memory_space=pl.ANY),
        ],
        out_specs=pl.BlockSpec(memory_space=pl.ANY),
        out_shape=jax.ShapeDtypeStruct((B, N_TOK, D), jnp.float32),
        input_output_aliases={2: 0},
        scratch_shapes=[pltpu.SemaphoreType.DMA] * 4,
    )(sel, tmpl, out)
    return out


def kernel(label, cluster, cls_ctx, token_prefix, token_suffix,
           token_prefix_cluster, token_suffix_cluster):
    sel = cluster.astype(jnp.int32)
    zmid = jnp.zeros((N_CLS, D), dtype=token_prefix.dtype)
    tmpl = jnp.stack([
        jnp.concatenate([token_prefix[0], zmid, token_suffix[0]], axis=0),
        jnp.concatenate([token_prefix_cluster[0], zmid,
                         token_suffix_cluster[0]], axis=0),
    ])
    return _call(tmpl, cls_ctx, label, sel)
